# Initial kernel scaffold; baseline (speedup 1.0000x reference)
import jax
import jax.numpy as jnp
from jax.experimental import pallas as pl


def kernel(edge_index, emb1):
    raise NotImplementedError("write your pallas kernel here")


# speedup vs baseline: 4.1376x; 4.1376x over previous
import functools

import jax
import jax.numpy as jnp
from jax import lax
from jax.experimental import pallas as pl
from jax.experimental.pallas import tpu as pltpu
from jax.experimental.pallas import tpu_sc as plsc

NC = 2
NS = 16
NW = NC * NS
LANES = 16

C = 128


def _build_sc_kernel(e_pad: int, n_nodes: int, dim: int):
    assert dim == 64
    nch = e_pad // (NW * C)
    mesh = plsc.VectorSubcoreMesh(core_axis_name="c", subcore_axis_name="s")

    @functools.partial(
        pl.kernel,
        out_type=jax.ShapeDtypeStruct((e_pad,), jnp.float32),
        mesh=mesh,
        compiler_params=pltpu.CompilerParams(
            needs_layout_passes=False, use_tc_tiling_on_sc=False),
        scratch_types=[
            pltpu.VMEM((C,), jnp.int32),
            pltpu.VMEM((C,), jnp.int32),
            pltpu.VMEM((C, 64), jnp.float32),
            pltpu.VMEM((C, 64), jnp.float32),
            pltpu.VMEM((16, LANES), jnp.float32),
            pltpu.VMEM((C,), jnp.float32),
            pltpu.SemaphoreType.DMA,
            pltpu.SemaphoreType.DMA,
        ],
    )
    def k(src_hbm, dst_hbm, emb_hbm, out_hbm,
          sidx, didx, srows, trows, part, outv, sem_s, sem_t):
        wid = lax.axis_index("s") * NC + lax.axis_index("c")
        lane = lax.iota(jnp.int32, LANES)

        def chunk_body(ci, _):
            row = wid * nch + ci
            pltpu.sync_copy(src_hbm.at[row], sidx)
            pltpu.sync_copy(dst_hbm.at[row], didx)
            cp_s = pltpu.async_copy(emb_hbm.at[sidx], srows, sem_s)
            cp_t = pltpu.async_copy(emb_hbm.at[didx], trows, sem_t)
            cp_s.wait()
            cp_t.wait()

            def group(g, _):
                e0 = g * 16
                for kk in range(16):
                    s = srows.at[e0 + kk]
                    t = trows.at[e0 + kk]
                    acc = (s[pl.ds(0, 16)] * t[pl.ds(0, 16)]
                           + s[pl.ds(16, 16)] * t[pl.ds(16, 16)]
                           + s[pl.ds(32, 16)] * t[pl.ds(32, 16)]
                           + s[pl.ds(48, 16)] * t[pl.ds(48, 16)])
                    part[kk, :] = acc
                tot = jnp.zeros((LANES,), jnp.float32)
                for j in range(16):
                    tot = tot + plsc.load_gather(
                        part, [lane, jnp.full((LANES,), j, jnp.int32)])
                outv[pl.ds(e0, 16)] = tot
                return 0

            lax.fori_loop(0, C // 16, group, 0)
            pltpu.sync_copy(outv, out_hbm.at[pl.ds(row * C, C)])
            return 0

        lax.fori_loop(0, nch, chunk_body, 0)

    return k


def kernel(edge_index, emb1):
    n_nodes, dim = emb1.shape
    e = edge_index.shape[1]
    block = NW * C
    e_pad = ((e + block - 1) // block) * block
    src = edge_index[0]
    dst = edge_index[1]
    if e_pad != e:
        pad = jnp.zeros((e_pad - e,), jnp.int32)
        src = jnp.concatenate([src, pad])
        dst = jnp.concatenate([dst, pad])
    nch = e_pad // block
    src2d = src.reshape(NW * nch, C)
    dst2d = dst.reshape(NW * nch, C)
    out = _build_sc_kernel(e_pad, n_nodes, dim)(src2d, dst2d, emb1)
    return out[:e]


# speedup vs baseline: 7.3234x; 1.7699x over previous
import functools

import jax
import jax.numpy as jnp
from jax import lax
from jax.experimental import pallas as pl
from jax.experimental.pallas import tpu as pltpu
from jax.experimental.pallas import tpu_sc as plsc

NC = 2
NS = 16
NW = NC * NS
LANES = 16

C = 128
NBUF = 2


def _build_sc_kernel(e_pad: int, n_nodes: int, dim: int):
    assert dim == 64
    nch = e_pad // (NW * C)
    assert nch % NBUF == 0
    mesh = plsc.VectorSubcoreMesh(core_axis_name="c", subcore_axis_name="s")

    @functools.partial(
        pl.kernel,
        out_type=jax.ShapeDtypeStruct((NW * nch, C), jnp.float32),
        mesh=mesh,
        compiler_params=pltpu.CompilerParams(
            needs_layout_passes=False, use_tc_tiling_on_sc=False),
        scratch_types=[
            pltpu.VMEM((nch, C), jnp.int32),
            pltpu.VMEM((nch, C), jnp.int32),
            pltpu.VMEM((C, 64), jnp.float32),
            pltpu.VMEM((C, 64), jnp.float32),
            pltpu.VMEM((C, 64), jnp.float32),
            pltpu.VMEM((C, 64), jnp.float32),
            pltpu.VMEM((16, LANES), jnp.float32),
            pltpu.VMEM((nch, C), jnp.float32),
            pltpu.SemaphoreType.DMA,
            pltpu.SemaphoreType.DMA,
            pltpu.SemaphoreType.DMA,
            pltpu.SemaphoreType.DMA,
        ],
    )
    def k(src_hbm, dst_hbm, emb_hbm, out_hbm,
          sidx, didx, srows0, trows0, srows1, trows1, part, outbuf,
          sem_s0, sem_t0, sem_s1, sem_t1):
        wid = lax.axis_index("s") * NC + lax.axis_index("c")
        lane = lax.iota(jnp.int32, LANES)
        srows = (srows0, srows1)
        trows = (trows0, trows1)
        sems = ((sem_s0, sem_t0), (sem_s1, sem_t1))

        pltpu.sync_copy(src_hbm.at[pl.ds(wid * nch, nch), :], sidx)
        pltpu.sync_copy(dst_hbm.at[pl.ds(wid * nch, nch), :], didx)

        def descs(b, c):
            return (
                pltpu.make_async_copy(emb_hbm.at[sidx.at[c]], srows[b], sems[b][0]),
                pltpu.make_async_copy(emb_hbm.at[didx.at[c]], trows[b], sems[b][1]),
            )

        for b in range(NBUF):
            for d in descs(b, b):
                d.start()

        def compute(sr, tr, c):
            def group(g, _):
                e0 = g * 16
                for kk in range(16):
                    s = sr.at[e0 + kk]
                    t = tr.at[e0 + kk]
                    acc = (s[pl.ds(0, 16)] * t[pl.ds(0, 16)]
                           + s[pl.ds(16, 16)] * t[pl.ds(16, 16)]
                           + s[pl.ds(32, 16)] * t[pl.ds(32, 16)]
                           + s[pl.ds(48, 16)] * t[pl.ds(48, 16)])
                    part[kk, :] = acc
                tot = jnp.zeros((LANES,), jnp.float32)
                for j in range(16):
                    tot = tot + plsc.load_gather(
                        part, [lane, jnp.full((LANES,), j, jnp.int32)])
                outbuf.at[c][pl.ds(e0, 16)] = tot
                return 0

            lax.fori_loop(0, C // 16, group, 0)

        def macro(m, _):
            for b in range(NBUF):
                c = m * NBUF + b
                for d in descs(b, c):
                    d.wait()
                compute(srows[b], trows[b], c)

                @pl.when(c + NBUF < nch)
                def _():
                    for d in descs(b, c + NBUF):
                        d.start()

            return 0

        lax.fori_loop(0, nch // NBUF, macro, 0)
        pltpu.sync_copy(outbuf, out_hbm.at[pl.ds(wid * nch, nch), :])

    return k


def kernel(edge_index, emb1):
    n_nodes, dim = emb1.shape
    e = edge_index.shape[1]
    block = NW * C * NBUF
    e_pad = ((e + block - 1) // block) * block
    src = edge_index[0]
    dst = edge_index[1]
    if e_pad != e:
        pad = jnp.zeros((e_pad - e,), jnp.int32)
        src = jnp.concatenate([src, pad])
        dst = jnp.concatenate([dst, pad])
    nch = e_pad // (NW * C)
    src2d = src.reshape(NW * nch, C)
    dst2d = dst.reshape(NW * nch, C)
    out = _build_sc_kernel(e_pad, n_nodes, dim)(src2d, dst2d, emb1)
    return out.reshape(e_pad)[:e]
